# baseline (speedup 1.0000x reference)
import functools

import jax
import jax.numpy as jnp
from jax.experimental import pallas as pl
from jax.experimental.pallas import tpu as pltpu

HIDDEN = 256
NUM_GRAPHS = 64
BN_EPS = 1e-5
N_NODES = 10000
N_EDGES = 160000

_NODE_BM = 1000
_EDGE_BM = 2000


def _proj_body(x_ref, w_ref, b_ref, o_ref):
    x = x_ref[...]
    x = jnp.where(x == x, x, 0.0)
    o_ref[...] = jax.nn.relu(
        jnp.dot(x, w_ref[...], preferred_element_type=jnp.float32) + b_ref[...]
    )


def _proj(x, w, b, bm):
    rows, k = x.shape
    h = w.shape[1]
    grid = rows // bm
    return pl.pallas_call(
        _proj_body,
        grid=(grid,),
        in_specs=[
            pl.BlockSpec((bm, k), lambda i: (i, 0)),
            pl.BlockSpec((k, h), lambda i: (0, 0)),
            pl.BlockSpec((1, h), lambda i: (0, 0)),
        ],
        out_specs=pl.BlockSpec((bm, h), lambda i: (i, 0)),
        out_shape=jax.ShapeDtypeStruct((rows, h), jnp.float32),
    )(x, w, b)


def _mlp_body(x_ref, a_ref, w1_ref, b1_ref, w2_ref, b2_ref, o_ref):
    h = x_ref[...] + a_ref[...]
    h = jax.nn.relu(
        jnp.dot(h, w1_ref[...], preferred_element_type=jnp.float32) + b1_ref[...]
    )
    h = jnp.dot(h, w2_ref[...], preferred_element_type=jnp.float32) + b2_ref[...]
    o_ref[...] = jax.nn.relu(h)


def _mlp(x, aggr, w1t, b1, w2t, b2, bm):
    rows = x.shape[0]
    grid = rows // bm
    h1 = w1t.shape[1]
    return pl.pallas_call(
        _mlp_body,
        grid=(grid,),
        in_specs=[
            pl.BlockSpec((bm, HIDDEN), lambda i: (i, 0)),
            pl.BlockSpec((bm, HIDDEN), lambda i: (i, 0)),
            pl.BlockSpec((HIDDEN, h1), lambda i: (0, 0)),
            pl.BlockSpec((1, h1), lambda i: (0, 0)),
            pl.BlockSpec((h1, HIDDEN), lambda i: (0, 0)),
            pl.BlockSpec((1, HIDDEN), lambda i: (0, 0)),
        ],
        out_specs=pl.BlockSpec((bm, HIDDEN), lambda i: (i, 0)),
        out_shape=jax.ShapeDtypeStruct((rows, HIDDEN), jnp.float32),
    )(x, aggr, w1t, b1, w2t, b2)


def _pool_body(x_ref, seg_ref, o_ref, sum_s, cnt_s):
    i = pl.program_id(0)

    @pl.when(i == 0)
    def _():
        sum_s[...] = jnp.zeros_like(sum_s)
        cnt_s[...] = jnp.zeros_like(cnt_s)

    seg = seg_ref[0, 0, :]
    bm = seg.shape[0]
    onehot = (
        seg[None, :]
        == jax.lax.broadcasted_iota(jnp.int32, (NUM_GRAPHS, bm), 0)
    ).astype(jnp.float32)
    sum_s[...] += jnp.dot(onehot, x_ref[...], preferred_element_type=jnp.float32)
    cnt_s[...] += jnp.dot(
        onehot, jnp.ones((bm, HIDDEN), jnp.float32),
        preferred_element_type=jnp.float32,
    )

    @pl.when(i == pl.num_programs(0) - 1)
    def _():
        o_ref[...] = sum_s[...] / jnp.maximum(cnt_s[...], 1.0)


def _pool(x, batch_ids, bm):
    rows = x.shape[0]
    grid = rows // bm
    seg3 = batch_ids.reshape(grid, 1, bm)
    return pl.pallas_call(
        _pool_body,
        grid=(grid,),
        in_specs=[
            pl.BlockSpec((bm, HIDDEN), lambda i: (i, 0)),
            pl.BlockSpec((1, 1, bm), lambda i: (i, 0, 0)),
        ],
        out_specs=pl.BlockSpec((NUM_GRAPHS, HIDDEN), lambda i: (0, 0)),
        out_shape=jax.ShapeDtypeStruct((NUM_GRAPHS, HIDDEN), jnp.float32),
        scratch_shapes=[
            pltpu.VMEM((NUM_GRAPHS, HIDDEN), jnp.float32),
            pltpu.VMEM((NUM_GRAPHS, HIDDEN), jnp.float32),
        ],
    )(x, seg3)


def _message_pass(x, e, src, dst):
    msg = jax.nn.relu(x[src] + e)
    return jax.ops.segment_sum(msg, dst, num_segments=N_NODES)


def _encoder(params, x_in, edge_index, edge_attr, batch_ids):
    npw_t = params['node_proj_w'].T
    epw_t = params['edge_proj_w'].T
    x = _proj(x_in, npw_t, params['node_proj_b'][None, :], _NODE_BM)
    e = _proj(edge_attr, epw_t, params['edge_proj_b'][None, :], _EDGE_BM)
    src = edge_index[0]
    dst = edge_index[1]
    for layer in params['layers']:
        scale = layer['bn_gamma'] * jax.lax.rsqrt(layer['bn_var'] + BN_EPS)
        shift = layer['bn_beta'] - layer['bn_mean'] * scale
        w1t = layer['w1'].T
        w2t = layer['w2'].T * scale[None, :]
        b2 = (layer['b2'] * scale + shift)[None, :]
        aggr = _message_pass(x, e, src, dst)
        x = _mlp(x, aggr, w1t, layer['b1'][None, :], w2t, b2, _NODE_BM)
    return _pool(x, batch_ids, _NODE_BM)


def kernel(prot_x, prot_edge_index, prot_edge_attr, prot_batch,
           drug_x, drug_edge_index, drug_edge_attr, drug_batch,
           prot_params, drug_params):
    p = _encoder(prot_params, prot_x, prot_edge_index, prot_edge_attr, prot_batch)
    d = _encoder(drug_params, drug_x, drug_edge_index, drug_edge_attr, drug_batch)
    return (p, d)
